# baseline (speedup 1.0000x reference)
import functools

import jax
import jax.numpy as jnp
from jax.experimental import pallas as pl
from jax.experimental.pallas import tpu as pltpu

_W = 16384
_NH = 2


def _loss_kernel(x_ref, t_ref, o_ref, sum_acc, tgt_acc, *, nblk, width, ncols):
    j = pl.program_id(1)
    x = x_ref[...]
    b, w = x.shape
    col = j * width + jax.lax.broadcasted_iota(jnp.int32, (b, w), 1)
    e = jnp.where(col < ncols, jnp.exp(x), 0.0)
    s = jnp.sum(e, axis=1, keepdims=True)
    tv = jnp.sum(jnp.where(col == t_ref[...], x, 0.0), axis=1, keepdims=True)

    @pl.when(j == 0)
    def _init():
        sum_acc[...] = s
        tgt_acc[...] = tv

    @pl.when(j > 0)
    def _accum():
        sum_acc[...] += s
        tgt_acc[...] += tv

    @pl.when(j == nblk - 1)
    def _finish():
        p = jnp.exp(tgt_acc[...]) / sum_acc[...]
        o_ref[...] = jnp.sum(jnp.log(p + 1e-6)).reshape(1, 1, 1)


def kernel(input, target):
    b, c = input.shape
    nblk = pl.cdiv(c, _W)
    bh = b // _NH
    t2 = target.astype(jnp.int32).reshape(b, 1)
    partials = pl.pallas_call(
        functools.partial(_loss_kernel, nblk=nblk, width=_W, ncols=c),
        grid=(_NH, nblk),
        in_specs=[
            pl.BlockSpec((bh, _W), lambda h, j: (h, j)),
            pl.BlockSpec((bh, 1), lambda h, j: (h, 0)),
        ],
        out_specs=pl.BlockSpec((1, 1, 1), lambda h, j: (h, 0, 0)),
        out_shape=jax.ShapeDtypeStruct((_NH, 1, 1), jnp.float32),
        scratch_shapes=[
            pltpu.VMEM((bh, 1), jnp.float32),
            pltpu.VMEM((bh, 1), jnp.float32),
        ],
        compiler_params=pltpu.CompilerParams(
            dimension_semantics=("parallel", "arbitrary"),
        ),
    )(input, t2)
    return -jnp.sum(partials) / b


# speedup vs baseline: 1.0836x; 1.0836x over previous
import functools

import jax
import jax.numpy as jnp
from jax.experimental import pallas as pl
from jax.experimental.pallas import tpu as pltpu

_W = 4096
_K = 4


def _loss_kernel(*refs, nsteps, width, ncols, nstreams):
    x_refs = refs[:nstreams]
    t_ref = refs[nstreams]
    o_ref = refs[nstreams + 1]
    sum_acc = refs[nstreams + 2]
    tgt_acc = refs[nstreams + 3]
    j = pl.program_id(0)

    s = None
    tv = None
    for k in range(nstreams):
        x = x_refs[k][...]
        b, w = x.shape
        col = (j * nstreams + k) * width + jax.lax.broadcasted_iota(
            jnp.int32, (b, w), 1
        )
        e = jnp.where(col < ncols, jnp.exp(x), 0.0)
        sk = jnp.sum(e, axis=1, keepdims=True)
        tk = jnp.sum(jnp.where(col == t_ref[...], x, 0.0), axis=1, keepdims=True)
        s = sk if s is None else s + sk
        tv = tk if tv is None else tv + tk

    @pl.when(j == 0)
    def _init():
        sum_acc[...] = s
        tgt_acc[...] = tv

    @pl.when(j > 0)
    def _accum():
        sum_acc[...] += s
        tgt_acc[...] += tv

    @pl.when(j == nsteps - 1)
    def _finish():
        p = jnp.exp(tgt_acc[...]) / sum_acc[...]
        o_ref[...] = -jnp.mean(jnp.log(p + 1e-6)).reshape(1, 1)


def kernel(input, target):
    b, c = input.shape
    nsteps = pl.cdiv(c, _W * _K)
    t2 = target.astype(jnp.int32).reshape(b, 1)

    nblocks = pl.cdiv(c, _W)

    def _x_spec(k):
        return pl.BlockSpec(
            (b, _W), lambda j, _k=k: (0, jnp.minimum(j * _K + _k, nblocks - 1))
        )

    out = pl.pallas_call(
        functools.partial(
            _loss_kernel, nsteps=nsteps, width=_W, ncols=c, nstreams=_K
        ),
        grid=(nsteps,),
        in_specs=[_x_spec(k) for k in range(_K)]
        + [pl.BlockSpec((b, 1), lambda j: (0, 0))],
        out_specs=pl.BlockSpec((1, 1), lambda j: (0, 0)),
        out_shape=jax.ShapeDtypeStruct((1, 1), jnp.float32),
        scratch_shapes=[
            pltpu.VMEM((b, 1), jnp.float32),
            pltpu.VMEM((b, 1), jnp.float32),
        ],
    )(*([input] * _K), t2)
    return out[0, 0]
